# baseline (speedup 1.0000x reference)
import jax
import jax.numpy as jnp
from jax.experimental import pallas as pl
from jax.experimental.pallas import tpu as pltpu

_SCORE_THRESH = 0.05
_NMS_THRESH = 0.5
_DETS = 100
_N = 20000
_LANES = 128
_ROWS = 160
_NPAD = _ROWS * _LANES
_NEG_INF = float("-inf")


def _argmax_first(v, idx):
    m1 = jnp.max(v, axis=0, keepdims=True)
    rcand = jnp.where(v == m1, idx, float(_NPAD))
    r1 = jnp.min(rcand, axis=0, keepdims=True)
    m = jnp.max(m1)
    sel = jnp.min(jnp.where(m1 == m, r1, float(_NPAD)))
    return m, sel


def _nms_body(v_ref, out_ref,
              bx1_ref, by1_ref, bx2_ref, by2_ref, area_ref, sw_ref):
    x1 = v_ref[0 * _ROWS:1 * _ROWS, :]
    y1 = v_ref[1 * _ROWS:2 * _ROWS, :]
    x2 = v_ref[2 * _ROWS:3 * _ROWS, :]
    y2 = v_ref[3 * _ROWS:4 * _ROWS, :]
    s = v_ref[4 * _ROWS:5 * _ROWS, :]
    cls_f = v_ref[5 * _ROWS:6 * _ROWS, :]

    max_c = jnp.maximum(jnp.maximum(jnp.max(x1), jnp.max(y1)),
                        jnp.maximum(jnp.max(x2), jnp.max(y2)))
    scale = max_c + 1.0
    offs = cls_f * scale
    bx1_ref[...] = x1 + offs
    by1_ref[...] = y1 + offs
    bx2_ref[...] = x2 + offs
    by2_ref[...] = y2 + offs
    w = jnp.maximum(bx2_ref[...] - bx1_ref[...], 0.0)
    h = jnp.maximum(by2_ref[...] - by1_ref[...], 0.0)
    area_ref[...] = w * h

    idx = (jax.lax.broadcasted_iota(jnp.int32, (_ROWS, _LANES), 0) * _LANES
           + jax.lax.broadcasted_iota(jnp.int32, (_ROWS, _LANES), 1)
           ).astype(jnp.float32)
    lane = jax.lax.broadcasted_iota(jnp.int32, (1, _LANES), 1)

    sw0 = jnp.where(s > _SCORE_THRESH, s, _NEG_INF)
    sw_ref[...] = sw0
    m0, sel0 = _argmax_first(sw0, idx)

    def step(i, carry):
        m, sel = carry
        sel_i = sel.astype(jnp.int32)
        r = jax.lax.shift_right_logical(sel_i, 7)
        l = jax.lax.bitwise_and(sel_i, _LANES - 1)
        lmask = lane == l

        def pick(row0):
            return jnp.max(jnp.where(lmask, v_ref[pl.ds(row0 + r, 1), :],
                                     _NEG_INF))

        ox1 = pick(0 * _ROWS)
        oy1 = pick(1 * _ROWS)
        ox2 = pick(2 * _ROWS)
        oy2 = pick(3 * _ROWS)
        cls_s = pick(5 * _ROWS)

        row = jnp.where(lane == 0, ox1,
              jnp.where(lane == 1, oy1,
              jnp.where(lane == 2, ox2,
              jnp.where(lane == 3, oy2,
              jnp.where(lane == 4, m, cls_s)))))
        out_ref[pl.ds(i, 1), :] = row

        offs_s = cls_s * scale
        sx1 = ox1 + offs_s
        sy1 = oy1 + offs_s
        sx2 = ox2 + offs_s
        sy2 = oy2 + offs_s
        area_s = jnp.maximum(sx2 - sx1, 0.0) * jnp.maximum(sy2 - sy1, 0.0)

        xx1 = jnp.maximum(bx1_ref[...], sx1)
        yy1 = jnp.maximum(by1_ref[...], sy1)
        xx2 = jnp.minimum(bx2_ref[...], sx2)
        yy2 = jnp.minimum(by2_ref[...], sy2)
        iw = jnp.maximum(xx2 - xx1, 0.0)
        ih = jnp.maximum(yy2 - yy1, 0.0)
        inter = iw * ih
        iou = inter / (area_ref[...] + area_s - inter + 1e-9)
        supp = (iou > _NMS_THRESH) | (idx == sel)
        s_new = jnp.where(supp, _NEG_INF, sw_ref[...])
        sw_ref[...] = s_new

        return _argmax_first(s_new, idx)

    jax.lax.fori_loop(0, _DETS, step, (m0, sel0), unroll=2)


def kernel(boxes, scores, classes):
    stacked = jnp.concatenate(
        [jnp.swapaxes(boxes, 0, 1),
         scores[None, :],
         classes.astype(jnp.float32)[None, :]], axis=0)
    padded = jnp.pad(stacked, ((0, 0), (0, _NPAD - _N)))

    out = pl.pallas_call(
        _nms_body,
        out_shape=jax.ShapeDtypeStruct((_DETS, _LANES), jnp.float32),
        in_specs=[pl.BlockSpec(memory_space=pltpu.VMEM)],
        scratch_shapes=[pltpu.VMEM((_ROWS, _LANES), jnp.float32)
                        for _ in range(6)],
    )(padded.reshape(6 * _ROWS, _LANES))

    kept_boxes = out[:, 0:4]
    kept_scores = out[:, 4]
    kept_classes = out[:, 5].astype(jnp.int32)
    return kept_boxes, kept_scores, kept_classes


# speedup vs baseline: 1.0288x; 1.0288x over previous
import jax
import jax.numpy as jnp
from jax.experimental import pallas as pl
from jax.experimental.pallas import tpu as pltpu

_SCORE_THRESH = 0.05
_NMS_THRESH = 0.5
_DETS = 100
_N = 20000
_LANES = 128
_ROWS = 160
_NPAD = _ROWS * _LANES
_NEG_INF = float("-inf")


def _argmax_first(v, idx):
    m1 = jnp.max(v, axis=0, keepdims=True)
    rcand = jnp.where(v == m1, idx, float(_NPAD))
    r1 = jnp.min(rcand, axis=0, keepdims=True)
    m = jnp.max(m1)
    sel = jnp.min(jnp.where(m1 == m, r1, float(_NPAD)))
    return m, sel


def _nms_body(v_ref, s_ref, out_ref,
              bx1_ref, by1_ref, bx2_ref, by2_ref, area_ref, sw_ref):
    x1 = v_ref[0 * _ROWS:1 * _ROWS, :]
    y1 = v_ref[1 * _ROWS:2 * _ROWS, :]
    x2 = v_ref[2 * _ROWS:3 * _ROWS, :]
    y2 = v_ref[3 * _ROWS:4 * _ROWS, :]
    cls_f = v_ref[4 * _ROWS:5 * _ROWS, :]
    s = v_ref[5 * _ROWS:6 * _ROWS, :]

    max_c = jnp.maximum(jnp.maximum(jnp.max(x1), jnp.max(y1)),
                        jnp.maximum(jnp.max(x2), jnp.max(y2)))
    scale = max_c + 1.0
    offs = cls_f * scale
    bx1_ref[...] = x1 + offs
    by1_ref[...] = y1 + offs
    bx2_ref[...] = x2 + offs
    by2_ref[...] = y2 + offs
    w = jnp.maximum(bx2_ref[...] - bx1_ref[...], 0.0)
    h = jnp.maximum(by2_ref[...] - by1_ref[...], 0.0)
    area_ref[...] = w * h

    idx = (jax.lax.broadcasted_iota(jnp.int32, (_ROWS, _LANES), 0) * _LANES
           + jax.lax.broadcasted_iota(jnp.int32, (_ROWS, _LANES), 1)
           ).astype(jnp.float32)
    lane = jax.lax.broadcasted_iota(jnp.int32, (1, _LANES), 1)

    sw0 = jnp.where(s > _SCORE_THRESH, s, _NEG_INF)
    sw_ref[...] = sw0
    m0, sel0 = _argmax_first(sw0, idx)

    def step(i, carry):
        m, sel = carry
        sel_i = sel.astype(jnp.int32)

        ox1 = s_ref[0, sel_i]
        oy1 = s_ref[1, sel_i]
        ox2 = s_ref[2, sel_i]
        oy2 = s_ref[3, sel_i]
        cls_s = s_ref[4, sel_i]

        row = jnp.where(lane == 0, ox1,
              jnp.where(lane == 1, oy1,
              jnp.where(lane == 2, ox2,
              jnp.where(lane == 3, oy2,
              jnp.where(lane == 4, m, cls_s)))))
        out_ref[pl.ds(i, 1), :] = row

        offs_s = cls_s * scale
        sx1 = ox1 + offs_s
        sy1 = oy1 + offs_s
        sx2 = ox2 + offs_s
        sy2 = oy2 + offs_s
        area_s = jnp.maximum(sx2 - sx1, 0.0) * jnp.maximum(sy2 - sy1, 0.0)

        xx1 = jnp.maximum(bx1_ref[...], sx1)
        yy1 = jnp.maximum(by1_ref[...], sy1)
        xx2 = jnp.minimum(bx2_ref[...], sx2)
        yy2 = jnp.minimum(by2_ref[...], sy2)
        iw = jnp.maximum(xx2 - xx1, 0.0)
        ih = jnp.maximum(yy2 - yy1, 0.0)
        inter = iw * ih
        iou = inter / (area_ref[...] + area_s - inter + 1e-9)
        supp = (iou > _NMS_THRESH) | (idx == sel)
        s_new = jnp.where(supp, _NEG_INF, sw_ref[...])
        sw_ref[...] = s_new

        return _argmax_first(s_new, idx)

    jax.lax.fori_loop(0, _DETS, step, (m0, sel0), unroll=4)


def kernel(boxes, scores, classes):
    stacked = jnp.concatenate(
        [jnp.swapaxes(boxes, 0, 1),
         classes.astype(jnp.float32)[None, :],
         scores[None, :]], axis=0)
    padded = jnp.pad(stacked, ((0, 0), (0, _NPAD - _N)))

    out = pl.pallas_call(
        _nms_body,
        out_shape=jax.ShapeDtypeStruct((_DETS, _LANES), jnp.float32),
        in_specs=[pl.BlockSpec(memory_space=pltpu.VMEM),
                  pl.BlockSpec(memory_space=pltpu.SMEM)],
        scratch_shapes=[pltpu.VMEM((_ROWS, _LANES), jnp.float32)
                        for _ in range(6)],
    )(padded.reshape(6 * _ROWS, _LANES), padded[:5])

    kept_boxes = out[:, 0:4]
    kept_scores = out[:, 4]
    kept_classes = out[:, 5].astype(jnp.int32)
    return kept_boxes, kept_scores, kept_classes
